# Initial kernel scaffold; baseline (speedup 1.0000x reference)
import jax
import jax.numpy as jnp
from jax.experimental import pallas as pl


def kernel(x, edge_index, W1, b1, W2, b2, gamma, beta, Wf1, bf1, Wf2, bf2):
    raise NotImplementedError("write your pallas kernel here")


# speedup vs baseline: 12.5756x; 12.5756x over previous
import functools

import jax
import jax.numpy as jnp
from jax import lax
from jax.experimental import pallas as pl
from jax.experimental.pallas import tpu as pltpu
from jax.experimental.pallas import tpu_sc as plsc

N = 10000
E = 320000
D = 128

NC = 2
NS = 16
NW = NC * NS
EPW = E // NW
K = 80
NCHUNK = EPW // K
RPT = 624
RPT_LAST = N - (NS - 1) * RPT


def _copy_tile_rows(s, copy_fn):
    base = s * RPT

    @pl.when(s < NS - 1)
    def _():
        copy_fn(base, RPT)

    @pl.when(s == NS - 1)
    def _():
        copy_fn(base, RPT_LAST)

_MESH = plsc.VectorSubcoreMesh(core_axis_name="c", subcore_axis_name="s")


@functools.partial(
    pl.kernel,
    out_type=jax.ShapeDtypeStruct((NC, N, D), jnp.float32),
    mesh=_MESH,
    scratch_types=[
        pltpu.VMEM((K,), jnp.int32),
        pltpu.VMEM((K, D), jnp.float32),
        pltpu.VMEM_SHARED((N, D), jnp.float32),
    ],
)
def _sc_degree(dst_hbm, half_hbm, ones_hbm, out_hbm, didx, ones_v, acc):
    c = lax.axis_index("c")
    s = lax.axis_index("s")
    wid = c * NS + s
    _copy_tile_rows(s, lambda b, n: pltpu.sync_copy(
        half_hbm.at[pl.ds(b, n), :], acc.at[pl.ds(b, n), :]))
    pltpu.sync_copy(ones_hbm, ones_v)
    plsc.subcore_barrier()

    def body(j, _):
        base = wid * EPW + j * K
        pltpu.sync_copy(dst_hbm.at[pl.ds(base, K)], didx)
        pltpu.sync_copy(ones_v, acc.at[didx], add=True)
        return ()

    lax.fori_loop(0, NCHUNK, body, (), unroll=False)
    plsc.subcore_barrier()
    _copy_tile_rows(s, lambda b, n: pltpu.sync_copy(
        acc.at[pl.ds(b, n), :], out_hbm.at[c, pl.ds(b, n), :]))


@functools.partial(
    pl.kernel,
    out_type=jax.ShapeDtypeStruct((NC, N, D), jnp.float32),
    mesh=_MESH,
    scratch_types=[
        pltpu.VMEM((K,), jnp.int32),
        pltpu.VMEM((K,), jnp.int32),
        pltpu.VMEM((K, D), jnp.float32),
        pltpu.VMEM_SHARED((N, D), jnp.float32),
        pltpu.SemaphoreType.DMA,
    ],
)
def _sc_scatter(g_hbm, src_hbm, dst_hbm, out_hbm, sidx, didx, rows, acc, sem):
    c = lax.axis_index("c")
    s = lax.axis_index("s")
    wid = c * NS + s
    _copy_tile_rows(s, lambda b, n: pltpu.sync_copy(
        g_hbm.at[pl.ds(b, n), :], acc.at[pl.ds(b, n), :]))
    plsc.subcore_barrier()

    def body(j, _):
        base = wid * EPW + j * K
        pltpu.sync_copy(src_hbm.at[pl.ds(base, K)], sidx)
        pltpu.sync_copy(dst_hbm.at[pl.ds(base, K)], didx)
        pltpu.async_copy(g_hbm.at[sidx], rows, sem).wait()
        pltpu.sync_copy(rows, acc.at[didx], add=True)
        return ()

    lax.fori_loop(0, NCHUNK, body, (), unroll=False)
    plsc.subcore_barrier()
    _copy_tile_rows(s, lambda b, n: pltpu.sync_copy(
        acc.at[pl.ds(b, n), :], out_hbm.at[c, pl.ds(b, n), :]))


def _tc1_body(x_ref, w1_ref, degp_ref, dinv_ref, g1_ref):
    deg = degp_ref[0, :, 0:1] + degp_ref[1, :, 0:1]
    dinv = lax.rsqrt(deg)
    h = jnp.dot(x_ref[...], w1_ref[...], preferred_element_type=jnp.float32)
    dinv_ref[...] = dinv
    g1_ref[...] = h * dinv


def _tc2_body(scatp_ref, g1_ref, dinv_ref, b1_ref, w2_ref, g2_ref):
    pre = scatp_ref[0] + scatp_ref[1] - g1_ref[...]
    h1 = jnp.maximum(dinv_ref[...] * pre + b1_ref[...], 0.0)
    h2 = jnp.dot(h1, w2_ref[...], preferred_element_type=jnp.float32)
    g2_ref[...] = h2 * dinv_ref[...]


def _tc3_body(scatp_ref, g2_ref, dinv_ref, b2_ref, gamma_ref, beta_ref,
              wf1_ref, bf1_ref, wf2_ref, bf2_ref, out_ref):
    pre = scatp_ref[0] + scatp_ref[1] - g2_ref[...]
    h = dinv_ref[...] * pre + b2_ref[...]
    h = jnp.where(h > 0, h, 0.01 * h)
    mu = jnp.mean(h, axis=0, keepdims=True)
    xc = h - mu
    var = jnp.mean(xc * xc, axis=0, keepdims=True)
    hn = gamma_ref[...] * xc / jnp.sqrt(var + 1e-5) + beta_ref[...]
    t = jnp.dot(hn, wf1_ref[...], preferred_element_type=jnp.float32)
    t = t + bf1_ref[...]
    t = jnp.where(t > 0, t, 0.01 * t)
    out_ref[...] = (jnp.dot(t, wf2_ref[...],
                            preferred_element_type=jnp.float32) + bf2_ref[...])


def kernel(x, edge_index, W1, b1, W2, b2, gamma, beta, Wf1, bf1, Wf2, bf2):
    f32 = jnp.float32
    src = edge_index[0]
    dst = edge_index[1]

    half = jnp.full((N, D), 0.5, f32)
    ones_blk = jnp.ones((K, D), f32)
    degp = _sc_degree(dst, half, ones_blk)

    dinv, g1 = pl.pallas_call(
        _tc1_body,
        out_shape=(jax.ShapeDtypeStruct((N, 1), f32),
                   jax.ShapeDtypeStruct((N, D), f32)),
    )(x, W1, degp)

    scatp1 = _sc_scatter(g1, src, dst)

    g2 = pl.pallas_call(
        _tc2_body,
        out_shape=jax.ShapeDtypeStruct((N, D), f32),
    )(scatp1, g1, dinv, b1.reshape(1, D), W2)

    scatp2 = _sc_scatter(g2, src, dst)

    Wf1p = jnp.zeros((D, 128), f32).at[:, :Wf1.shape[1]].set(Wf1)
    bf1p = jnp.zeros((1, 128), f32).at[0, :bf1.shape[0]].set(bf1)
    Wf2p = jnp.zeros((128, 128), f32).at[:Wf2.shape[0], :Wf2.shape[1]].set(Wf2)
    bf2p = jnp.zeros((1, 128), f32).at[0, :bf2.shape[0]].set(bf2)

    out128 = pl.pallas_call(
        _tc3_body,
        out_shape=jax.ShapeDtypeStruct((N, 128), f32),
    )(scatp2, g2, dinv, b2.reshape(1, D), gamma.reshape(1, D),
      beta.reshape(1, D), Wf1p, bf1p, Wf2p, bf2p)

    return out128[:, :Wf2.shape[1]]


# speedup vs baseline: 22.4434x; 1.7847x over previous
import functools

import jax
import jax.numpy as jnp
from jax import lax
from jax.experimental import pallas as pl
from jax.experimental.pallas import tpu as pltpu
from jax.experimental.pallas import tpu_sc as plsc

N = 10000
E = 320000
D = 128

NC = 2
NS = 16
NW = NC * NS
EPW = E // NW
K = 80
NCHUNK = EPW // K
RPT = 624
RPT_LAST = N - (NS - 1) * RPT


def _copy_tile_rows(s, copy_fn):
    base = s * RPT

    @pl.when(s < NS - 1)
    def _():
        copy_fn(base, RPT)

    @pl.when(s == NS - 1)
    def _():
        copy_fn(base, RPT_LAST)

_MESH = plsc.VectorSubcoreMesh(core_axis_name="c", subcore_axis_name="s")


@functools.partial(
    pl.kernel,
    out_type=jax.ShapeDtypeStruct((NC, N, D), jnp.float32),
    mesh=_MESH,
    scratch_types=[
        pltpu.VMEM((K,), jnp.int32),
        pltpu.VMEM((K,), jnp.int32),
        pltpu.VMEM((K, D), jnp.float32),
        pltpu.VMEM_SHARED((N, D), jnp.float32),
        pltpu.SemaphoreType.DMA,
        pltpu.SemaphoreType.DMA,
    ],
)
def _sc_degree(dst_hbm, half_hbm, ones_hbm, out_hbm, didx0, didx1, ones_v,
               acc, isem0, isem1):
    c = lax.axis_index("c")
    s = lax.axis_index("s")
    wid = c * NS + s
    ebase = wid * EPW
    didx = (didx0, didx1)
    isem = (isem0, isem1)

    def issue_idx(jc, b):
        pltpu.async_copy(dst_hbm.at[pl.ds(ebase + jc * K, K)], didx[b],
                         isem[b])

    def wait_idx(b):
        pltpu.make_async_copy(dst_hbm.at[pl.ds(0, K)], didx[b],
                              isem[b]).wait()

    def scatter(b):
        pltpu.sync_copy(ones_v, acc.at[didx[b]], add=True)

    _copy_tile_rows(s, lambda b, n: pltpu.sync_copy(
        half_hbm.at[pl.ds(b, n), :], acc.at[pl.ds(b, n), :]))
    pltpu.sync_copy(ones_hbm, ones_v)
    plsc.subcore_barrier()

    issue_idx(0, 0)
    issue_idx(1, 1)

    def body(j2, _):
        a = 2 * j2
        wait_idx(0)
        scatter(0)

        @pl.when(a + 2 < NCHUNK)
        def _():
            issue_idx(a + 2, 0)

        @pl.when(a + 1 < NCHUNK)
        def _():
            wait_idx(1)
            scatter(1)

        @pl.when(a + 3 < NCHUNK)
        def _():
            issue_idx(a + 3, 1)

        return ()

    lax.fori_loop(0, (NCHUNK + 1) // 2, body, (), unroll=False)
    plsc.subcore_barrier()
    _copy_tile_rows(s, lambda b, n: pltpu.sync_copy(
        acc.at[pl.ds(b, n), :], out_hbm.at[c, pl.ds(b, n), :]))


@functools.partial(
    pl.kernel,
    out_type=jax.ShapeDtypeStruct((NC, N, D), jnp.float32),
    mesh=_MESH,
    scratch_types=[
        pltpu.VMEM((K,), jnp.int32),
        pltpu.VMEM((K,), jnp.int32),
        pltpu.VMEM((K,), jnp.int32),
        pltpu.VMEM((K,), jnp.int32),
        pltpu.VMEM((K, D), jnp.float32),
        pltpu.VMEM((K, D), jnp.float32),
        pltpu.VMEM_SHARED((N, D), jnp.float32),
        pltpu.SemaphoreType.DMA,
        pltpu.SemaphoreType.DMA,
        pltpu.SemaphoreType.DMA,
        pltpu.SemaphoreType.DMA,
    ],
)
def _sc_scatter(g_hbm, src_hbm, dst_hbm, out_hbm, sidx0, sidx1, didx0, didx1,
                rows0, rows1, acc, isem0, isem1, gsem0, gsem1):
    c = lax.axis_index("c")
    s = lax.axis_index("s")
    wid = c * NS + s
    ebase = wid * EPW
    sidx = (sidx0, sidx1)
    didx = (didx0, didx1)
    rows = (rows0, rows1)
    isem = (isem0, isem1)
    gsem = (gsem0, gsem1)

    def issue_idx(jc, b):
        base = ebase + jc * K
        pltpu.async_copy(src_hbm.at[pl.ds(base, K)], sidx[b], isem[b])
        pltpu.async_copy(dst_hbm.at[pl.ds(base, K)], didx[b], isem[b])

    def wait_idx(b):
        pltpu.make_async_copy(src_hbm.at[pl.ds(0, K)], sidx[b], isem[b]).wait()
        pltpu.make_async_copy(dst_hbm.at[pl.ds(0, K)], didx[b], isem[b]).wait()

    def issue_gather(b):
        pltpu.async_copy(g_hbm.at[sidx[b]], rows[b], gsem[b])

    def wait_gather(b):
        pltpu.make_async_copy(g_hbm.at[pl.ds(0, K), :], rows[b],
                              gsem[b]).wait()

    def scatter(b):
        pltpu.sync_copy(rows[b], acc.at[didx[b]], add=True)

    _copy_tile_rows(s, lambda b, n: pltpu.sync_copy(
        g_hbm.at[pl.ds(b, n), :], acc.at[pl.ds(b, n), :]))
    plsc.subcore_barrier()

    issue_idx(0, 0)
    wait_idx(0)
    issue_gather(0)
    issue_idx(1, 1)

    def body(j2, _):
        a = 2 * j2
        wait_gather(0)

        @pl.when(a + 1 < NCHUNK)
        def _():
            wait_idx(1)
            issue_gather(1)

        scatter(0)

        @pl.when(a + 2 < NCHUNK)
        def _():
            issue_idx(a + 2, 0)

        @pl.when(a + 1 < NCHUNK)
        def _():
            wait_gather(1)

        @pl.when(a + 2 < NCHUNK)
        def _():
            wait_idx(0)
            issue_gather(0)

        @pl.when(a + 1 < NCHUNK)
        def _():
            scatter(1)

        @pl.when(a + 3 < NCHUNK)
        def _():
            issue_idx(a + 3, 1)

        return ()

    lax.fori_loop(0, (NCHUNK + 1) // 2, body, (), unroll=False)
    plsc.subcore_barrier()
    _copy_tile_rows(s, lambda b, n: pltpu.sync_copy(
        acc.at[pl.ds(b, n), :], out_hbm.at[c, pl.ds(b, n), :]))


def _tc1_body(x_ref, w1_ref, degp_ref, dinv_ref, g1_ref):
    deg = degp_ref[0, :, 0:1] + degp_ref[1, :, 0:1]
    dinv = lax.rsqrt(deg)
    h = jnp.dot(x_ref[...], w1_ref[...], preferred_element_type=jnp.float32)
    dinv_ref[...] = dinv
    g1_ref[...] = h * dinv


def _tc2_body(scatp_ref, g1_ref, dinv_ref, b1_ref, w2_ref, g2_ref):
    pre = scatp_ref[0] + scatp_ref[1] - g1_ref[...]
    h1 = jnp.maximum(dinv_ref[...] * pre + b1_ref[...], 0.0)
    h2 = jnp.dot(h1, w2_ref[...], preferred_element_type=jnp.float32)
    g2_ref[...] = h2 * dinv_ref[...]


def _tc3_body(scatp_ref, g2_ref, dinv_ref, b2_ref, gamma_ref, beta_ref,
              wf1_ref, bf1_ref, wf2_ref, bf2_ref, out_ref):
    pre = scatp_ref[0] + scatp_ref[1] - g2_ref[...]
    h = dinv_ref[...] * pre + b2_ref[...]
    h = jnp.where(h > 0, h, 0.01 * h)
    mu = jnp.mean(h, axis=0, keepdims=True)
    xc = h - mu
    var = jnp.mean(xc * xc, axis=0, keepdims=True)
    hn = gamma_ref[...] * xc / jnp.sqrt(var + 1e-5) + beta_ref[...]
    t = jnp.dot(hn, wf1_ref[...], preferred_element_type=jnp.float32)
    t = t + bf1_ref[...]
    t = jnp.where(t > 0, t, 0.01 * t)
    out_ref[...] = (jnp.dot(t, wf2_ref[...],
                            preferred_element_type=jnp.float32) + bf2_ref[...])


def kernel(x, edge_index, W1, b1, W2, b2, gamma, beta, Wf1, bf1, Wf2, bf2):
    f32 = jnp.float32
    src = edge_index[0]
    dst = edge_index[1]

    half = jnp.full((N, D), 0.5, f32)
    ones_blk = jnp.ones((K, D), f32)
    degp = _sc_degree(dst, half, ones_blk)

    dinv, g1 = pl.pallas_call(
        _tc1_body,
        out_shape=(jax.ShapeDtypeStruct((N, 1), f32),
                   jax.ShapeDtypeStruct((N, D), f32)),
    )(x, W1, degp)

    scatp1 = _sc_scatter(g1, src, dst)

    g2 = pl.pallas_call(
        _tc2_body,
        out_shape=jax.ShapeDtypeStruct((N, D), f32),
    )(scatp1, g1, dinv, b1.reshape(1, D), W2)

    scatp2 = _sc_scatter(g2, src, dst)

    Wf1p = jnp.zeros((D, 128), f32).at[:, :Wf1.shape[1]].set(Wf1)
    bf1p = jnp.zeros((1, 128), f32).at[0, :bf1.shape[0]].set(bf1)
    Wf2p = jnp.zeros((128, 128), f32).at[:Wf2.shape[0], :Wf2.shape[1]].set(Wf2)
    bf2p = jnp.zeros((1, 128), f32).at[0, :bf2.shape[0]].set(bf2)

    out128 = pl.pallas_call(
        _tc3_body,
        out_shape=jax.ShapeDtypeStruct((N, 128), f32),
    )(scatp2, g2, dinv, b2.reshape(1, D), gamma.reshape(1, D),
      beta.reshape(1, D), Wf1p, bf1p, Wf2p, bf2p)

    return out128[:, :Wf2.shape[1]]


# speedup vs baseline: 25.3611x; 1.1300x over previous
import functools

import jax
import jax.numpy as jnp
from jax import lax
from jax.experimental import pallas as pl
from jax.experimental.pallas import tpu as pltpu
from jax.experimental.pallas import tpu_sc as plsc

N = 10000
E = 320000
D = 128

NC = 2
NS = 16
NW = NC * NS
K = 128
TCH = E // K
CQ = TCH // NW
CR = TCH % NW


def _chunk_range(wid):
    start = wid * CQ + jnp.minimum(wid, CR)
    cnt = CQ + jnp.where(wid < CR, 1, 0)
    return start, cnt
RPT = 624
RPT_LAST = N - (NS - 1) * RPT


def _copy_tile_rows(s, copy_fn):
    base = s * RPT

    @pl.when(s < NS - 1)
    def _():
        copy_fn(base, RPT)

    @pl.when(s == NS - 1)
    def _():
        copy_fn(base, RPT_LAST)

_MESH = plsc.VectorSubcoreMesh(core_axis_name="c", subcore_axis_name="s")


@functools.partial(
    pl.kernel,
    out_type=jax.ShapeDtypeStruct((NC, N, D), jnp.float32),
    mesh=_MESH,
    scratch_types=[
        pltpu.VMEM((K,), jnp.int32),
        pltpu.VMEM((K,), jnp.int32),
        pltpu.VMEM((K, D), jnp.float32),
        pltpu.VMEM_SHARED((N, D), jnp.float32),
        pltpu.SemaphoreType.DMA,
        pltpu.SemaphoreType.DMA,
    ],
)
def _sc_degree(dst_hbm, half_hbm, ones_hbm, out_hbm, didx0, didx1, ones_v,
               acc, isem0, isem1):
    c = lax.axis_index("c")
    s = lax.axis_index("s")
    wid = c * NS + s
    cstart, cnt = _chunk_range(wid)
    didx = (didx0, didx1)
    isem = (isem0, isem1)

    def issue_idx(jc, b):
        pltpu.async_copy(dst_hbm.at[pl.ds((cstart + jc) * K, K)], didx[b],
                         isem[b])

    def wait_idx(b):
        pltpu.make_async_copy(dst_hbm.at[pl.ds(0, K)], didx[b],
                              isem[b]).wait()

    def scatter(b):
        pltpu.sync_copy(ones_v, acc.at[didx[b]], add=True)

    _copy_tile_rows(s, lambda b, n: pltpu.sync_copy(
        half_hbm.at[pl.ds(b, n), :], acc.at[pl.ds(b, n), :]))
    pltpu.sync_copy(ones_hbm, ones_v)
    plsc.subcore_barrier()

    issue_idx(0, 0)
    issue_idx(1, 1)

    def body(j2, _):
        a = 2 * j2
        wait_idx(0)
        scatter(0)

        @pl.when(a + 2 < cnt)
        def _():
            issue_idx(a + 2, 0)

        @pl.when(a + 1 < cnt)
        def _():
            wait_idx(1)
            scatter(1)

        @pl.when(a + 3 < cnt)
        def _():
            issue_idx(a + 3, 1)

        return ()

    lax.fori_loop(0, (cnt + 1) // 2, body, (), unroll=False)
    plsc.subcore_barrier()
    _copy_tile_rows(s, lambda b, n: pltpu.sync_copy(
        acc.at[pl.ds(b, n), :], out_hbm.at[c, pl.ds(b, n), :]))


@functools.partial(
    pl.kernel,
    out_type=jax.ShapeDtypeStruct((NC, N, D), jnp.float32),
    mesh=_MESH,
    scratch_types=[
        pltpu.VMEM((K,), jnp.int32),
        pltpu.VMEM((K,), jnp.int32),
        pltpu.VMEM((K,), jnp.int32),
        pltpu.VMEM((K,), jnp.int32),
        pltpu.VMEM((K, D), jnp.float32),
        pltpu.VMEM((K, D), jnp.float32),
        pltpu.VMEM_SHARED((N, D), jnp.float32),
        pltpu.SemaphoreType.DMA,
        pltpu.SemaphoreType.DMA,
        pltpu.SemaphoreType.DMA,
        pltpu.SemaphoreType.DMA,
    ],
)
def _sc_scatter(g_hbm, src_hbm, dst_hbm, out_hbm, sidx0, sidx1, didx0, didx1,
                rows0, rows1, acc, isem0, isem1, gsem0, gsem1):
    c = lax.axis_index("c")
    s = lax.axis_index("s")
    wid = c * NS + s
    cstart, cnt = _chunk_range(wid)
    sidx = (sidx0, sidx1)
    didx = (didx0, didx1)
    rows = (rows0, rows1)
    isem = (isem0, isem1)
    gsem = (gsem0, gsem1)

    def issue_idx(jc, b):
        base = (cstart + jc) * K
        pltpu.async_copy(src_hbm.at[pl.ds(base, K)], sidx[b], isem[b])
        pltpu.async_copy(dst_hbm.at[pl.ds(base, K)], didx[b], isem[b])

    def wait_idx(b):
        pltpu.make_async_copy(src_hbm.at[pl.ds(0, K)], sidx[b], isem[b]).wait()
        pltpu.make_async_copy(dst_hbm.at[pl.ds(0, K)], didx[b], isem[b]).wait()

    def issue_gather(b):
        pltpu.async_copy(g_hbm.at[sidx[b]], rows[b], gsem[b])

    def wait_gather(b):
        pltpu.make_async_copy(g_hbm.at[pl.ds(0, K), :], rows[b],
                              gsem[b]).wait()

    def scatter(b):
        pltpu.sync_copy(rows[b], acc.at[didx[b]], add=True)

    _copy_tile_rows(s, lambda b, n: pltpu.sync_copy(
        g_hbm.at[pl.ds(b, n), :], acc.at[pl.ds(b, n), :]))
    plsc.subcore_barrier()

    issue_idx(0, 0)
    wait_idx(0)
    issue_gather(0)
    issue_idx(1, 1)

    def body(j2, _):
        a = 2 * j2
        wait_gather(0)

        @pl.when(a + 1 < cnt)
        def _():
            wait_idx(1)
            issue_gather(1)

        scatter(0)

        @pl.when(a + 2 < cnt)
        def _():
            issue_idx(a + 2, 0)

        @pl.when(a + 1 < cnt)
        def _():
            wait_gather(1)

        @pl.when(a + 2 < cnt)
        def _():
            wait_idx(0)
            issue_gather(0)

        @pl.when(a + 1 < cnt)
        def _():
            scatter(1)

        @pl.when(a + 3 < cnt)
        def _():
            issue_idx(a + 3, 1)

        return ()

    lax.fori_loop(0, (cnt + 1) // 2, body, (), unroll=False)
    plsc.subcore_barrier()
    _copy_tile_rows(s, lambda b, n: pltpu.sync_copy(
        acc.at[pl.ds(b, n), :], out_hbm.at[c, pl.ds(b, n), :]))


def _tc1_body(x_ref, w1_ref, degp_ref, dinv_ref, g1_ref):
    deg = degp_ref[0, :, 0:1] + degp_ref[1, :, 0:1]
    dinv = lax.rsqrt(deg)
    h = jnp.dot(x_ref[...], w1_ref[...], preferred_element_type=jnp.float32)
    dinv_ref[...] = dinv
    g1_ref[...] = h * dinv


def _tc2_body(scatp_ref, g1_ref, dinv_ref, b1_ref, w2_ref, g2_ref):
    pre = scatp_ref[0] + scatp_ref[1] - g1_ref[...]
    h1 = jnp.maximum(dinv_ref[...] * pre + b1_ref[...], 0.0)
    h2 = jnp.dot(h1, w2_ref[...], preferred_element_type=jnp.float32)
    g2_ref[...] = h2 * dinv_ref[...]


def _tc3_body(scatp_ref, g2_ref, dinv_ref, b2_ref, gamma_ref, beta_ref,
              wf1_ref, bf1_ref, wf2_ref, bf2_ref, out_ref):
    pre = scatp_ref[0] + scatp_ref[1] - g2_ref[...]
    h = dinv_ref[...] * pre + b2_ref[...]
    h = jnp.where(h > 0, h, 0.01 * h)
    mu = jnp.mean(h, axis=0, keepdims=True)
    xc = h - mu
    var = jnp.mean(xc * xc, axis=0, keepdims=True)
    hn = gamma_ref[...] * xc / jnp.sqrt(var + 1e-5) + beta_ref[...]
    t = jnp.dot(hn, wf1_ref[...], preferred_element_type=jnp.float32)
    t = t + bf1_ref[...]
    t = jnp.where(t > 0, t, 0.01 * t)
    out_ref[...] = (jnp.dot(t, wf2_ref[...],
                            preferred_element_type=jnp.float32) + bf2_ref[...])


def kernel(x, edge_index, W1, b1, W2, b2, gamma, beta, Wf1, bf1, Wf2, bf2):
    f32 = jnp.float32
    src = edge_index[0]
    dst = edge_index[1]

    half = jnp.full((N, D), 0.5, f32)
    ones_blk = jnp.ones((K, D), f32)
    degp = _sc_degree(dst, half, ones_blk)

    dinv, g1 = pl.pallas_call(
        _tc1_body,
        out_shape=(jax.ShapeDtypeStruct((N, 1), f32),
                   jax.ShapeDtypeStruct((N, D), f32)),
    )(x, W1, degp)

    scatp1 = _sc_scatter(g1, src, dst)

    g2 = pl.pallas_call(
        _tc2_body,
        out_shape=jax.ShapeDtypeStruct((N, D), f32),
    )(scatp1, g1, dinv, b1.reshape(1, D), W2)

    scatp2 = _sc_scatter(g2, src, dst)

    Wf1p = jnp.zeros((D, 128), f32).at[:, :Wf1.shape[1]].set(Wf1)
    bf1p = jnp.zeros((1, 128), f32).at[0, :bf1.shape[0]].set(bf1)
    Wf2p = jnp.zeros((128, 128), f32).at[:Wf2.shape[0], :Wf2.shape[1]].set(Wf2)
    bf2p = jnp.zeros((1, 128), f32).at[0, :bf2.shape[0]].set(bf2)

    out128 = pl.pallas_call(
        _tc3_body,
        out_shape=jax.ShapeDtypeStruct((N, 128), f32),
    )(scatp2, g2, dinv, b2.reshape(1, D), gamma.reshape(1, D),
      beta.reshape(1, D), Wf1p, bf1p, Wf2p, bf2p)

    return out128[:, :Wf2.shape[1]]


# speedup vs baseline: 27.0686x; 1.0673x over previous
import functools

import jax
import jax.numpy as jnp
from jax import lax
from jax.experimental import pallas as pl
from jax.experimental.pallas import tpu as pltpu
from jax.experimental.pallas import tpu_sc as plsc

N = 10000
E = 320000
D = 128

NC = 2
NS = 16
NW = NC * NS
K = 128
TCH = E // K
CQ = TCH // NW
CR = TCH % NW


def _chunk_range(wid):
    start = wid * CQ + jnp.minimum(wid, CR)
    cnt = CQ + jnp.where(wid < CR, 1, 0)
    return start, cnt
RPT = 624
RPT_LAST = N - (NS - 1) * RPT


def _copy_tile_rows(s, copy_fn):
    base = s * RPT

    @pl.when(s < NS - 1)
    def _():
        copy_fn(base, RPT)

    @pl.when(s == NS - 1)
    def _():
        copy_fn(base, RPT_LAST)

_MESH = plsc.VectorSubcoreMesh(core_axis_name="c", subcore_axis_name="s")


@functools.partial(
    pl.kernel,
    out_type=jax.ShapeDtypeStruct((NC, N, D), jnp.float32),
    mesh=_MESH,
    scratch_types=[
        pltpu.VMEM((K,), jnp.int32),
        pltpu.VMEM((K,), jnp.int32),
        pltpu.VMEM((K, D), jnp.float32),
        pltpu.VMEM_SHARED((N, D), jnp.float32),
        pltpu.SemaphoreType.DMA,
        pltpu.SemaphoreType.DMA,
    ],
)
def _sc_degree(dst_hbm, half_hbm, ones_hbm, out_hbm, didx0, didx1, ones_v,
               acc, isem0, isem1):
    c = lax.axis_index("c")
    s = lax.axis_index("s")
    wid = c * NS + s
    cstart, cnt = _chunk_range(wid)
    didx = (didx0, didx1)
    isem = (isem0, isem1)

    def issue_idx(jc, b):
        pltpu.async_copy(dst_hbm.at[pl.ds((cstart + jc) * K, K)], didx[b],
                         isem[b])

    def wait_idx(b):
        pltpu.make_async_copy(dst_hbm.at[pl.ds(0, K)], didx[b],
                              isem[b]).wait()

    def scatter(b):
        pltpu.sync_copy(ones_v, acc.at[didx[b]], add=True)

    _copy_tile_rows(s, lambda b, n: pltpu.sync_copy(
        half_hbm.at[pl.ds(b, n), :], acc.at[pl.ds(b, n), :]))
    pltpu.sync_copy(ones_hbm, ones_v)
    plsc.subcore_barrier()

    issue_idx(0, 0)
    issue_idx(1, 1)

    def body(j2, _):
        a = 2 * j2
        wait_idx(0)
        scatter(0)

        @pl.when(a + 2 < cnt)
        def _():
            issue_idx(a + 2, 0)

        @pl.when(a + 1 < cnt)
        def _():
            wait_idx(1)
            scatter(1)

        @pl.when(a + 3 < cnt)
        def _():
            issue_idx(a + 3, 1)

        return ()

    lax.fori_loop(0, (cnt + 1) // 2, body, (), unroll=False)
    plsc.subcore_barrier()
    _copy_tile_rows(s, lambda b, n: pltpu.sync_copy(
        acc.at[pl.ds(b, n), :], out_hbm.at[c, pl.ds(b, n), :]))


@functools.partial(
    pl.kernel,
    out_type=jax.ShapeDtypeStruct((NC, N, D), jnp.float32),
    mesh=_MESH,
    scratch_types=[
        pltpu.VMEM((K,), jnp.int32),
        pltpu.VMEM((K,), jnp.int32),
        pltpu.VMEM((K,), jnp.int32),
        pltpu.VMEM((K,), jnp.int32),
        pltpu.VMEM((K,), jnp.int32),
        pltpu.VMEM((K,), jnp.int32),
        pltpu.VMEM((K, D), jnp.float32),
        pltpu.VMEM((K, D), jnp.float32),
        pltpu.VMEM((K, D), jnp.float32),
        pltpu.VMEM_SHARED((N, D), jnp.float32),
        pltpu.SemaphoreType.DMA,
        pltpu.SemaphoreType.DMA,
        pltpu.SemaphoreType.DMA,
        pltpu.SemaphoreType.DMA,
        pltpu.SemaphoreType.DMA,
        pltpu.SemaphoreType.DMA,
        pltpu.SemaphoreType.DMA,
        pltpu.SemaphoreType.DMA,
        pltpu.SemaphoreType.DMA,
    ],
)
def _sc_scatter(g_hbm, src_hbm, dst_hbm, out_hbm, sidx0, sidx1, sidx2,
                didx0, didx1, didx2, rows0, rows1, rows2, acc,
                isem0, isem1, isem2, gsem0, gsem1, gsem2,
                ssem0, ssem1, ssem2):
    c = lax.axis_index("c")
    s = lax.axis_index("s")
    wid = c * NS + s
    cstart, cnt = _chunk_range(wid)
    sidx = (sidx0, sidx1, sidx2)
    didx = (didx0, didx1, didx2)
    rows = (rows0, rows1, rows2)
    isem = (isem0, isem1, isem2)
    gsem = (gsem0, gsem1, gsem2)
    ssem = (ssem0, ssem1, ssem2)

    def issue_idx(jc, b):
        base = (cstart + jc) * K
        pltpu.async_copy(src_hbm.at[pl.ds(base, K)], sidx[b], isem[b])
        pltpu.async_copy(dst_hbm.at[pl.ds(base, K)], didx[b], isem[b])

    def wait_idx(b):
        pltpu.make_async_copy(src_hbm.at[pl.ds(0, K)], sidx[b], isem[b]).wait()
        pltpu.make_async_copy(dst_hbm.at[pl.ds(0, K)], didx[b], isem[b]).wait()

    def issue_gather(b):
        pltpu.async_copy(g_hbm.at[sidx[b]], rows[b], gsem[b])

    def wait_gather(b):
        pltpu.make_async_copy(g_hbm.at[pl.ds(0, K), :], rows[b],
                              gsem[b]).wait()

    def issue_scatter(b):
        pltpu.async_copy(rows[b], acc.at[didx[b]], ssem[b], add=True)

    def wait_scatter(b):
        pltpu.make_async_copy(g_hbm.at[pl.ds(0, K), :], rows[b],
                              ssem[b]).wait()

    _copy_tile_rows(s, lambda b, n: pltpu.sync_copy(
        g_hbm.at[pl.ds(b, n), :], acc.at[pl.ds(b, n), :]))
    plsc.subcore_barrier()

    for o in range(3):
        issue_idx(o, o)
    for o in range(3):
        wait_idx(o)
        issue_gather(o)

    def body(j3, _):
        a = 3 * j3
        for o in range(3):
            q = a + o

            @pl.when(q < cnt)
            def _(o=o):
                wait_gather(o)
                issue_scatter(o)

        for o in range(3):
            q2 = a + o + 3

            @pl.when(q2 < cnt)
            def _(o=o, q2=q2):
                wait_scatter(o)
                issue_idx(q2, o)
                wait_idx(o)
                issue_gather(o)

        return ()

    lax.fori_loop(0, (cnt + 2) // 3, body, (), unroll=False)
    for o in range(3):
        wait_scatter(o)
    plsc.subcore_barrier()
    _copy_tile_rows(s, lambda b, n: pltpu.sync_copy(
        acc.at[pl.ds(b, n), :], out_hbm.at[c, pl.ds(b, n), :]))


def _tc1_body(x_ref, w1_ref, degp_ref, dinv_ref, g1_ref):
    deg = degp_ref[0, :, 0:1] + degp_ref[1, :, 0:1]
    dinv = lax.rsqrt(deg)
    h = jnp.dot(x_ref[...], w1_ref[...], preferred_element_type=jnp.float32)
    dinv_ref[...] = dinv
    g1_ref[...] = h * dinv


def _tc2_body(scatp_ref, g1_ref, dinv_ref, b1_ref, w2_ref, g2_ref):
    pre = scatp_ref[0] + scatp_ref[1] - g1_ref[...]
    h1 = jnp.maximum(dinv_ref[...] * pre + b1_ref[...], 0.0)
    h2 = jnp.dot(h1, w2_ref[...], preferred_element_type=jnp.float32)
    g2_ref[...] = h2 * dinv_ref[...]


def _tc3_body(scatp_ref, g2_ref, dinv_ref, b2_ref, gamma_ref, beta_ref,
              wf1_ref, bf1_ref, wf2_ref, bf2_ref, out_ref):
    pre = scatp_ref[0] + scatp_ref[1] - g2_ref[...]
    h = dinv_ref[...] * pre + b2_ref[...]
    h = jnp.where(h > 0, h, 0.01 * h)
    mu = jnp.mean(h, axis=0, keepdims=True)
    xc = h - mu
    var = jnp.mean(xc * xc, axis=0, keepdims=True)
    hn = gamma_ref[...] * xc / jnp.sqrt(var + 1e-5) + beta_ref[...]
    t = jnp.dot(hn, wf1_ref[...], preferred_element_type=jnp.float32)
    t = t + bf1_ref[...]
    t = jnp.where(t > 0, t, 0.01 * t)
    out_ref[...] = (jnp.dot(t, wf2_ref[...],
                            preferred_element_type=jnp.float32) + bf2_ref[...])


def kernel(x, edge_index, W1, b1, W2, b2, gamma, beta, Wf1, bf1, Wf2, bf2):
    f32 = jnp.float32
    src = edge_index[0]
    dst = edge_index[1]

    half = jnp.full((N, D), 0.5, f32)
    ones_blk = jnp.ones((K, D), f32)
    degp = _sc_degree(dst, half, ones_blk)

    dinv, g1 = pl.pallas_call(
        _tc1_body,
        out_shape=(jax.ShapeDtypeStruct((N, 1), f32),
                   jax.ShapeDtypeStruct((N, D), f32)),
    )(x, W1, degp)

    scatp1 = _sc_scatter(g1, src, dst)

    g2 = pl.pallas_call(
        _tc2_body,
        out_shape=jax.ShapeDtypeStruct((N, D), f32),
    )(scatp1, g1, dinv, b1.reshape(1, D), W2)

    scatp2 = _sc_scatter(g2, src, dst)

    Wf1p = jnp.zeros((D, 128), f32).at[:, :Wf1.shape[1]].set(Wf1)
    bf1p = jnp.zeros((1, 128), f32).at[0, :bf1.shape[0]].set(bf1)
    Wf2p = jnp.zeros((128, 128), f32).at[:Wf2.shape[0], :Wf2.shape[1]].set(Wf2)
    bf2p = jnp.zeros((1, 128), f32).at[0, :bf2.shape[0]].set(bf2)

    out128 = pl.pallas_call(
        _tc3_body,
        out_shape=jax.ShapeDtypeStruct((N, 128), f32),
    )(scatp2, g2, dinv, b2.reshape(1, D), gamma.reshape(1, D),
      beta.reshape(1, D), Wf1p, bf1p, Wf2p, bf2p)

    return out128[:, :Wf2.shape[1]]
